# speedup vs baseline: 13.1405x; 13.1405x over previous
import functools

import jax
import jax.numpy as jnp
from jax import lax
from jax.experimental import pallas as pl
from jax.experimental.pallas import tpu as pltpu

N_PW = 8
EMBED = 128
LIN_IN = 170
ROWS = 512


def _body(ints_ref, wt_ref, b_ref, out_ref):
    ints = ints_ref[0]
    parts = []

    def seg(width, col, shift):
        i = lax.broadcasted_iota(jnp.int32, (ROWS, width), 1)
        return (i == (ints[:, col:col + 1] + shift)).astype(jnp.float32)

    parts.append(seg(9, 0, 1))
    parts.append(seg(10, 1, 0))
    parts.append(seg(7, 2, 0))
    for p in range(N_PW):
        parts.append(seg(10, 3 + 2 * p + 1, 1))
    for p in range(N_PW):
        parts.append(seg(8, 3 + 2 * p, 1))
    onehot = jnp.concatenate(parts, axis=1)
    acc = lax.dot_general(onehot, wt_ref[...],
                          (((1,), (0,)), ((), ())),
                          preferred_element_type=jnp.float32)
    out_ref[0] = acc + b_ref[...]


def kernel(weather, time_left, min_time_left, pseudo_weather, W, b):
    B, T = weather.shape
    N = B * T
    G = N // ROWS
    ints = jnp.concatenate(
        [weather.reshape(N, 1), time_left.reshape(N, 1),
         min_time_left.reshape(N, 1), pseudo_weather.reshape(N, 2 * N_PW)],
        axis=1).reshape(G, ROWS, 3 + 2 * N_PW)
    wt = W.T
    b2 = b.reshape(1, EMBED)

    out = pl.pallas_call(
        _body,
        grid=(G,),
        in_specs=[
            pl.BlockSpec((1, ROWS, 3 + 2 * N_PW), lambda g: (g, 0, 0)),
            pl.BlockSpec((LIN_IN, EMBED), lambda g: (0, 0)),
            pl.BlockSpec((1, EMBED), lambda g: (0, 0)),
        ],
        out_specs=pl.BlockSpec((1, ROWS, EMBED), lambda g: (g, 0, 0)),
        out_shape=jax.ShapeDtypeStruct((G, ROWS, EMBED), jnp.float32),
    )(ints, wt, b2)
    return out.reshape(B, T, EMBED)
